# Initial kernel scaffold; baseline (speedup 1.0000x reference)
import jax
import jax.numpy as jnp
from jax.experimental import pallas as pl


def kernel(seq_len, table):
    raise NotImplementedError("write your pallas kernel here")


# speedup vs baseline: 12.3837x; 12.3837x over previous
import functools

import jax
import jax.numpy as jnp
from jax import lax
from jax.experimental import pallas as pl
from jax.experimental.pallas import tpu as pltpu
from jax.experimental.pallas import tpu_sc as plsc

D_MODEL = 128
MAX_LEN = 512
SEQ_LEN = 512
TBL = 2 * MAX_LEN - 1


def _sc_materialize(t2):
    info = plsc.get_sparse_core_info()
    nw = info.num_cores * info.num_subcores
    rows = SEQ_LEN // nw
    mesh = plsc.VectorSubcoreMesh(core_axis_name="c", subcore_axis_name="s")

    @functools.partial(
        pl.kernel,
        mesh=mesh,
        out_type=jax.ShapeDtypeStruct((SEQ_LEN, SEQ_LEN, D_MODEL), jnp.float32),
        scratch_types=[
            pltpu.VMEM((TBL, D_MODEL), jnp.float32),
            pltpu.SemaphoreType.DMA,
        ],
    )
    def k(t2_hbm, out_hbm, t2_v, sem):
        wid = lax.axis_index("s") * info.num_cores + lax.axis_index("c")
        pltpu.sync_copy(t2_hbm, t2_v)
        base = wid * rows
        copies = []
        for r in range(rows):
            i = base + r
            copies.append(
                pltpu.async_copy(
                    t2_v.at[pl.ds(SEQ_LEN - 1 - i, SEQ_LEN)], out_hbm.at[i], sem
                )
            )
        for c in copies:
            c.wait()

    return k(t2)


def kernel(seq_len, table):
    delta = seq_len - SEQ_LEN
    t2 = table[jnp.clip(TBL - 1 + delta - jnp.arange(TBL), 0, TBL - 1)]
    return _sc_materialize(t2)


# speedup vs baseline: 13.9311x; 1.1250x over previous
import functools

import jax
import jax.numpy as jnp
from jax import lax
from jax.experimental import pallas as pl
from jax.experimental.pallas import tpu as pltpu
from jax.experimental.pallas import tpu_sc as plsc

D_MODEL = 128
MAX_LEN = 512
SEQ_LEN = 512
TBL = 2 * MAX_LEN - 1


def _sc_materialize(t2):
    info = plsc.get_sparse_core_info()
    nw = info.num_cores * info.num_subcores
    rows = SEQ_LEN // nw
    mesh = plsc.VectorSubcoreMesh(core_axis_name="c", subcore_axis_name="s")

    win = SEQ_LEN + rows

    @functools.partial(
        pl.kernel,
        mesh=mesh,
        out_type=jax.ShapeDtypeStruct((SEQ_LEN, SEQ_LEN, D_MODEL), jnp.float32),
        scratch_types=[
            pltpu.VMEM((win, D_MODEL), jnp.float32),
            pltpu.SemaphoreType.DMA,
        ],
    )
    def k(t2_hbm, out_hbm, win_v, sem):
        wid = lax.axis_index("s") * info.num_cores + lax.axis_index("c")
        base = wid * rows
        pltpu.sync_copy(t2_hbm.at[pl.ds(SEQ_LEN - rows - base, win)], win_v)
        copies = []
        for r in range(rows):
            copies.append(
                pltpu.async_copy(
                    win_v.at[pl.ds(rows - 1 - r, SEQ_LEN)], out_hbm.at[base + r], sem
                )
            )
        for c in copies:
            c.wait()

    return k(t2)


def kernel(seq_len, table):
    delta = seq_len - SEQ_LEN
    t2 = table[jnp.clip(TBL - 1 + delta - jnp.arange(TBL + 1), 0, TBL - 1)]
    return _sc_materialize(t2)
